# Initial kernel scaffold; baseline (speedup 1.0000x reference)
import functools

import jax
import jax.numpy as jnp
from jax import lax
from jax.experimental import pallas as pl
from jax.experimental.pallas import tpu as pltpu
from jax.experimental.pallas import tpu_sc as plsc

EPS = 1e-7

NC = 2
NS = 16
NW = NC * NS
VNUM = 10000
D = 128
E = 320000
EDGES_PER_W = E // NW
CHUNK = 80
NCHUNK = EDGES_PER_W // CHUNK
ROWS_PER_TILE = VNUM // NS
CP_BLK = 125


def _sc_body(sidx_hbm, tidx_hbm, enorm_hbm, esgn_hbm, vrepr_hbm, out_hbm,
             ptr_acc, sidx_v, tidx_v, w_v, tmp_v, rows_v, stage_v):
    cid = lax.axis_index("c")
    sid = lax.axis_index("s")
    wid = cid * NS + sid

    def _zero_row(r, _):
        for c8 in range(D // 16):
            stage_v[r, pl.ds(c8 * 16, 16)] = jnp.zeros((16,), jnp.float32)
        return _
    lax.fori_loop(0, CP_BLK, _zero_row, 0)
    for b in range(ROWS_PER_TILE // CP_BLK):
        pltpu.sync_copy(stage_v,
                        ptr_acc.at[pl.ds(sid * ROWS_PER_TILE + b * CP_BLK,
                                         CP_BLK)])
    plsc.subcore_barrier()

    pltpu.sync_copy(sidx_hbm.at[wid], sidx_v)
    pltpu.sync_copy(tidx_hbm.at[wid], tidx_v)
    pltpu.sync_copy(enorm_hbm.at[wid], w_v)
    pltpu.sync_copy(esgn_hbm.at[wid], tmp_v)

    def _wmul(r, _):
        for k in range(CHUNK // 16):
            sl = pl.ds(k * 16, 16)
            w_v[r, sl] = w_v[r, sl] * tmp_v[r, sl]
        return _
    lax.fori_loop(0, NCHUNK, _wmul, 0)

    def _chunk(j, _):
        pltpu.sync_copy(vrepr_hbm.at[sidx_v.at[j]], rows_v)

        def _scale(i, _c):
            w = w_v[j, i]
            for c8 in range(D // 16):
                sl = pl.ds(c8 * 16, 16)
                rows_v[i, sl] = rows_v[i, sl] * w
            return _c
        lax.fori_loop(0, CHUNK, _scale, 0)

        pltpu.sync_copy(rows_v, ptr_acc.at[tidx_v.at[j]], add=True)
        return _
    lax.fori_loop(0, NCHUNK, _chunk, 0)

    plsc.subcore_barrier()

    for b in range(ROWS_PER_TILE // CP_BLK):
        base = sid * ROWS_PER_TILE + b * CP_BLK
        pltpu.sync_copy(ptr_acc.at[pl.ds(base, CP_BLK)], stage_v)
        pltpu.sync_copy(stage_v, out_hbm.at[cid, pl.ds(base, CP_BLK)])


def _segment_sum_sc(sidx_r, tidx_r, enorm_r, esgn_r, vrepr):
    mesh = plsc.VectorSubcoreMesh(core_axis_name="c", subcore_axis_name="s",
                                  num_cores=NC, num_subcores=NS)
    return pl.kernel(
        _sc_body,
        out_type=jax.ShapeDtypeStruct((NC, VNUM, D), jnp.float32),
        mesh=mesh,
        scratch_types=[
            pltpu.VMEM_SHARED((VNUM, D), jnp.float32),
            pltpu.VMEM((NCHUNK, CHUNK), jnp.int32),
            pltpu.VMEM((NCHUNK, CHUNK), jnp.int32),
            pltpu.VMEM((NCHUNK, CHUNK), jnp.float32),
            pltpu.VMEM((NCHUNK, CHUNK), jnp.float32),
            pltpu.VMEM((CHUNK, D), jnp.float32),
            pltpu.VMEM((CP_BLK, D), jnp.float32),
        ],
    )(sidx_r, tidx_r, enorm_r, esgn_r, vrepr)


def _tc_body(a_ref, b_ref, wl_ref, bl_ref, ws_ref, bs_ref, loc_ref, std_ref):
    ptr = a_ref[0] + b_ref[0]
    zl = jnp.dot(ptr, wl_ref[...], preferred_element_type=jnp.float32)
    zs = jnp.dot(ptr, ws_ref[...], preferred_element_type=jnp.float32)
    loc_ref[...] = zl + bl_ref[...]
    std_ref[...] = jax.nn.softplus(zs + bs_ref[...]) + EPS


def _heads_tc(partials, W_loc, b_loc, W_std, b_std):
    blk = 1000
    grid = (VNUM // blk,)
    return pl.pallas_call(
        _tc_body,
        grid=grid,
        in_specs=[
            pl.BlockSpec((1, blk, D), lambda i: (0, i, 0)),
            pl.BlockSpec((1, blk, D), lambda i: (1, i, 0)),
            pl.BlockSpec((D, D), lambda i: (0, 0)),
            pl.BlockSpec((D,), lambda i: (0,)),
            pl.BlockSpec((D, D), lambda i: (0, 0)),
            pl.BlockSpec((D,), lambda i: (0,)),
        ],
        out_specs=[
            pl.BlockSpec((blk, D), lambda i: (i, 0)),
            pl.BlockSpec((blk, D), lambda i: (i, 0)),
        ],
        out_shape=[
            jax.ShapeDtypeStruct((VNUM, D), jnp.float32),
            jax.ShapeDtypeStruct((VNUM, D), jnp.float32),
        ],
    )(partials, partials, W_loc, b_loc, W_std, b_std)


def kernel(eidx, enorm, esgn, vrepr, W_loc, b_loc, W_std, b_std):
    sidx_r = eidx[0].astype(jnp.int32).reshape(NW, NCHUNK, CHUNK)
    tidx_r = eidx[1].astype(jnp.int32).reshape(NW, NCHUNK, CHUNK)
    enorm_r = enorm.reshape(NW, NCHUNK, CHUNK)
    esgn_r = esgn.reshape(NW, NCHUNK, CHUNK)
    partials = _segment_sum_sc(sidx_r, tidx_r, enorm_r, esgn_r, vrepr)
    loc, std = _heads_tc(partials, W_loc, b_loc, W_std, b_std)
    return (loc, std)


# speedup vs baseline: 4.8769x; 4.8769x over previous
import functools

import jax
import jax.numpy as jnp
from jax import lax
from jax.experimental import pallas as pl
from jax.experimental.pallas import tpu as pltpu
from jax.experimental.pallas import tpu_sc as plsc

EPS = 1e-7

NC = 2
NS = 16
NW = NC * NS
VNUM = 10000
D = 128
E = 320000
EDGES_PER_W = E // NW
CHUNK = 80
NCHUNK = EDGES_PER_W // CHUNK
NROWBLK = VNUM // CHUNK
BLK_ITERS = -(-NROWBLK // NS)


def _sc_body(sidx_hbm, tidx_hbm, enorm_hbm, esgn_hbm, vrepr_hbm, out_hbm,
             ptr_acc, sidx_v, tidx_v, en_v, es_v, rows_v):
    cid = lax.axis_index("c")
    sid = lax.axis_index("s")
    wid = cid * NS + sid

    def _zero_row(r, _):
        for c8 in range(D // 16):
            rows_v[r, pl.ds(c8 * 16, 16)] = jnp.zeros((16,), jnp.float32)
        return _
    lax.fori_loop(0, CHUNK, _zero_row, 0)
    for b in range(BLK_ITERS):
        blk = sid + NS * b

        @pl.when(blk < NROWBLK)
        def _():
            pltpu.sync_copy(rows_v, ptr_acc.at[pl.ds(blk * CHUNK, CHUNK)])
    plsc.subcore_barrier()

    pltpu.sync_copy(sidx_hbm.at[wid], sidx_v)
    pltpu.sync_copy(tidx_hbm.at[wid], tidx_v)

    def _chunk(j, _):
        pltpu.sync_copy(vrepr_hbm.at[sidx_v.at[j]], rows_v)
        pltpu.sync_copy(enorm_hbm.at[wid, j], en_v)
        pltpu.sync_copy(esgn_hbm.at[wid, j], es_v)

        for g in range(CHUNK // 16):
            sl = pl.ds(g * 16, 16)
            es_v[sl] = es_v[sl] * en_v[sl]

        for g in range(CHUNK // 16):
            wv = es_v[pl.ds(g * 16, 16)]
            for l in range(16):
                i = g * 16 + l
                w = wv[l]
                for c8 in range(D // 16):
                    sl = pl.ds(c8 * 16, 16)
                    rows_v[i, sl] = rows_v[i, sl] * w

        pltpu.sync_copy(rows_v, ptr_acc.at[tidx_v.at[j]], add=True)
        return _
    lax.fori_loop(0, NCHUNK, _chunk, 0)

    plsc.subcore_barrier()

    for b in range(BLK_ITERS):
        blk = sid + NS * b

        @pl.when(blk < NROWBLK)
        def _():
            pltpu.sync_copy(ptr_acc.at[pl.ds(blk * CHUNK, CHUNK)], rows_v)
            pltpu.sync_copy(rows_v, out_hbm.at[cid, pl.ds(blk * CHUNK, CHUNK)])


def _segment_sum_sc(sidx_r, tidx_r, enorm_r, esgn_r, vrepr):
    mesh = plsc.VectorSubcoreMesh(core_axis_name="c", subcore_axis_name="s",
                                  num_cores=NC, num_subcores=NS)
    return pl.kernel(
        _sc_body,
        out_type=jax.ShapeDtypeStruct((NC, VNUM, D), jnp.float32),
        mesh=mesh,
        scratch_types=[
            pltpu.VMEM_SHARED((VNUM, D), jnp.float32),
            pltpu.VMEM((NCHUNK, CHUNK), jnp.int32),
            pltpu.VMEM((NCHUNK, CHUNK), jnp.int32),
            pltpu.VMEM((CHUNK,), jnp.float32),
            pltpu.VMEM((CHUNK,), jnp.float32),
            pltpu.VMEM((CHUNK, D), jnp.float32),
        ],
    )(sidx_r, tidx_r, enorm_r, esgn_r, vrepr)


def _tc_body(a_ref, b_ref, wl_ref, bl_ref, ws_ref, bs_ref, loc_ref, std_ref):
    ptr = a_ref[0] + b_ref[0]
    zl = jnp.dot(ptr, wl_ref[...], preferred_element_type=jnp.float32)
    zs = jnp.dot(ptr, ws_ref[...], preferred_element_type=jnp.float32)
    loc_ref[...] = zl + bl_ref[...]
    std_ref[...] = jax.nn.softplus(zs + bs_ref[...]) + EPS


def _heads_tc(partials, W_loc, b_loc, W_std, b_std):
    blk = 1000
    grid = (VNUM // blk,)
    return pl.pallas_call(
        _tc_body,
        grid=grid,
        in_specs=[
            pl.BlockSpec((1, blk, D), lambda i: (0, i, 0)),
            pl.BlockSpec((1, blk, D), lambda i: (1, i, 0)),
            pl.BlockSpec((D, D), lambda i: (0, 0)),
            pl.BlockSpec((D,), lambda i: (0,)),
            pl.BlockSpec((D, D), lambda i: (0, 0)),
            pl.BlockSpec((D,), lambda i: (0,)),
        ],
        out_specs=[
            pl.BlockSpec((blk, D), lambda i: (i, 0)),
            pl.BlockSpec((blk, D), lambda i: (i, 0)),
        ],
        out_shape=[
            jax.ShapeDtypeStruct((VNUM, D), jnp.float32),
            jax.ShapeDtypeStruct((VNUM, D), jnp.float32),
        ],
    )(partials, partials, W_loc, b_loc, W_std, b_std)


def kernel(eidx, enorm, esgn, vrepr, W_loc, b_loc, W_std, b_std):
    sidx_r = eidx[0].astype(jnp.int32).reshape(NW, NCHUNK, CHUNK)
    tidx_r = eidx[1].astype(jnp.int32).reshape(NW, NCHUNK, CHUNK)
    enorm_r = enorm.reshape(NW, NCHUNK, CHUNK)
    esgn_r = esgn.reshape(NW, NCHUNK, CHUNK)
    partials = _segment_sum_sc(sidx_r, tidx_r, enorm_r, esgn_r, vrepr)
    loc, std = _heads_tc(partials, W_loc, b_loc, W_std, b_std)
    return (loc, std)


# speedup vs baseline: 10.4124x; 2.1350x over previous
import functools

import jax
import jax.numpy as jnp
from jax import lax
from jax.experimental import pallas as pl
from jax.experimental.pallas import tpu as pltpu
from jax.experimental.pallas import tpu_sc as plsc

EPS = 1e-7

NC = 2
NS = 16
NW = NC * NS
VNUM = 10000
D = 128
E = 320000
EDGES_PER_W = E // NW
CHUNK = 80
NCHUNK = EDGES_PER_W // CHUNK
NROWBLK = VNUM // CHUNK
BLK_ITERS = -(-NROWBLK // NS)


def _sc_body(sidx_hbm, tidx_hbm, enorm_hbm, esgn_hbm, vrepr_hbm, out_hbm,
             ptr_acc, sidx_v, tid0_v, tid1_v, en0_v, en1_v, es0_v, es1_v,
             rows0_v, rows1_v, sem0, sem1):
    cid = lax.axis_index("c")
    sid = lax.axis_index("s")
    wid = cid * NS + sid

    def _zero_row(r, _):
        for c8 in range(D // 16):
            rows0_v[r, pl.ds(c8 * 16, 16)] = jnp.zeros((16,), jnp.float32)
        return _
    lax.fori_loop(0, CHUNK, _zero_row, 0)
    for b in range(BLK_ITERS):
        blk = sid + NS * b

        @pl.when(blk < NROWBLK)
        def _():
            pltpu.sync_copy(rows0_v, ptr_acc.at[pl.ds(blk * CHUNK, CHUNK)])
    plsc.subcore_barrier()

    pltpu.sync_copy(sidx_hbm.at[wid], sidx_v)

    rows_b = (rows0_v, rows1_v)
    tid_b = (tid0_v, tid1_v)
    en_b = (en0_v, en1_v)
    es_b = (es0_v, es1_v)
    sem_b = (sem0, sem1)

    def _fire(j, b):
        pltpu.async_copy(vrepr_hbm.at[sidx_v.at[j]], rows_b[b], sem_b[b])
        pltpu.async_copy(tidx_hbm.at[wid, j], tid_b[b], sem_b[b])
        pltpu.async_copy(enorm_hbm.at[wid, j], en_b[b], sem_b[b])
        pltpu.async_copy(esgn_hbm.at[wid, j], es_b[b], sem_b[b])

    def _drain(b):
        pltpu.make_async_copy(vrepr_hbm.at[sidx_v.at[0]], rows_b[b],
                              sem_b[b]).wait()
        pltpu.make_async_copy(tidx_hbm.at[wid, 0], tid_b[b], sem_b[b]).wait()
        pltpu.make_async_copy(enorm_hbm.at[wid, 0], en_b[b], sem_b[b]).wait()
        pltpu.make_async_copy(esgn_hbm.at[wid, 0], es_b[b], sem_b[b]).wait()

    _fire(0, 0)
    _fire(1, 1)

    def _pair(g, carry):
        for b in range(2):
            j = 2 * g + b

            @pl.when(j < NCHUNK)
            def _():
                _drain(b)
                rows_v = rows_b[b]
                en_v = en_b[b]
                es_v = es_b[b]

                for q in range(CHUNK // 16):
                    sl = pl.ds(q * 16, 16)
                    es_v[sl] = es_v[sl] * en_v[sl]

                for q in range(CHUNK // 16):
                    wv = es_v[pl.ds(q * 16, 16)]
                    for l in range(16):
                        i = q * 16 + l
                        w = wv[l]
                        for c8 in range(D // 16):
                            sl = pl.ds(c8 * 16, 16)
                            rows_v[i, sl] = rows_v[i, sl] * w

                pltpu.sync_copy(rows_v, ptr_acc.at[tid_b[b]], add=True)

                @pl.when(j + 2 < NCHUNK)
                def _():
                    _fire(j + 2, b)
        return carry
    lax.fori_loop(0, (NCHUNK + 1) // 2, _pair, 0)

    plsc.subcore_barrier()

    for b in range(BLK_ITERS):
        blk = sid + NS * b

        @pl.when(blk < NROWBLK)
        def _():
            pltpu.sync_copy(ptr_acc.at[pl.ds(blk * CHUNK, CHUNK)], rows0_v)
            pltpu.sync_copy(rows0_v,
                            out_hbm.at[cid, pl.ds(blk * CHUNK, CHUNK)])


def _segment_sum_sc(sidx_r, tidx_r, enorm_r, esgn_r, vrepr):
    mesh = plsc.VectorSubcoreMesh(core_axis_name="c", subcore_axis_name="s",
                                  num_cores=NC, num_subcores=NS)
    return pl.kernel(
        _sc_body,
        out_type=jax.ShapeDtypeStruct((NC, VNUM, D), jnp.float32),
        mesh=mesh,
        scratch_types=[
            pltpu.VMEM_SHARED((VNUM, D), jnp.float32),
            pltpu.VMEM((NCHUNK, CHUNK), jnp.int32),
            pltpu.VMEM((CHUNK,), jnp.int32),
            pltpu.VMEM((CHUNK,), jnp.int32),
            pltpu.VMEM((CHUNK,), jnp.float32),
            pltpu.VMEM((CHUNK,), jnp.float32),
            pltpu.VMEM((CHUNK,), jnp.float32),
            pltpu.VMEM((CHUNK,), jnp.float32),
            pltpu.VMEM((CHUNK, D), jnp.float32),
            pltpu.VMEM((CHUNK, D), jnp.float32),
            pltpu.SemaphoreType.DMA,
            pltpu.SemaphoreType.DMA,
        ],
    )(sidx_r, tidx_r, enorm_r, esgn_r, vrepr)


def _tc_body(a_ref, b_ref, wl_ref, bl_ref, ws_ref, bs_ref, loc_ref, std_ref):
    ptr = a_ref[0] + b_ref[0]
    zl = jnp.dot(ptr, wl_ref[...], preferred_element_type=jnp.float32)
    zs = jnp.dot(ptr, ws_ref[...], preferred_element_type=jnp.float32)
    loc_ref[...] = zl + bl_ref[...]
    std_ref[...] = jax.nn.softplus(zs + bs_ref[...]) + EPS


def _heads_tc(partials, W_loc, b_loc, W_std, b_std):
    blk = 1000
    grid = (VNUM // blk,)
    return pl.pallas_call(
        _tc_body,
        grid=grid,
        in_specs=[
            pl.BlockSpec((1, blk, D), lambda i: (0, i, 0)),
            pl.BlockSpec((1, blk, D), lambda i: (1, i, 0)),
            pl.BlockSpec((D, D), lambda i: (0, 0)),
            pl.BlockSpec((D,), lambda i: (0,)),
            pl.BlockSpec((D, D), lambda i: (0, 0)),
            pl.BlockSpec((D,), lambda i: (0,)),
        ],
        out_specs=[
            pl.BlockSpec((blk, D), lambda i: (i, 0)),
            pl.BlockSpec((blk, D), lambda i: (i, 0)),
        ],
        out_shape=[
            jax.ShapeDtypeStruct((VNUM, D), jnp.float32),
            jax.ShapeDtypeStruct((VNUM, D), jnp.float32),
        ],
    )(partials, partials, W_loc, b_loc, W_std, b_std)


def kernel(eidx, enorm, esgn, vrepr, W_loc, b_loc, W_std, b_std):
    sidx_r = eidx[0].astype(jnp.int32).reshape(NW, NCHUNK, CHUNK)
    tidx_r = eidx[1].astype(jnp.int32).reshape(NW, NCHUNK, CHUNK)
    enorm_r = enorm.reshape(NW, NCHUNK, CHUNK)
    esgn_r = esgn.reshape(NW, NCHUNK, CHUNK)
    partials = _segment_sum_sc(sidx_r, tidx_r, enorm_r, esgn_r, vrepr)
    loc, std = _heads_tc(partials, W_loc, b_loc, W_std, b_std)
    return (loc, std)


# speedup vs baseline: 11.3837x; 1.0933x over previous
import functools

import jax
import jax.numpy as jnp
from jax import lax
from jax.experimental import pallas as pl
from jax.experimental.pallas import tpu as pltpu
from jax.experimental.pallas import tpu_sc as plsc

EPS = 1e-7

NC = 2
NS = 16
NW = NC * NS
VNUM = 10000
D = 128
E = 320000
EDGES_PER_W = E // NW
CHUNK = 80
NCHUNK = EDGES_PER_W // CHUNK
NROWBLK = VNUM // CHUNK
BLK_ITERS = -(-NROWBLK // NS)


def _sc_body(sidx_hbm, tidx_hbm, enorm_hbm, esgn_hbm, vrepr_hbm, out_hbm,
             ptr_acc, sid_ring, tid0_v, tid1_v, tst0_v, tst1_v,
             en0_v, en1_v, es0_v, es1_v,
             rows0_v, rows1_v, rs0_v, rs1_v,
             sem0, sem1, ssem0, ssem1, isem0, isem1):
    cid = lax.axis_index("c")
    sid = lax.axis_index("s")
    wid = cid * NS + sid

    def _zero_row(r, _):
        for c8 in range(D // 16):
            rows0_v[r, pl.ds(c8 * 16, 16)] = jnp.zeros((16,), jnp.float32)
        return _
    lax.fori_loop(0, CHUNK, _zero_row, 0)
    for b in range(BLK_ITERS):
        blk = sid + NS * b

        @pl.when(blk < NROWBLK)
        def _():
            pltpu.sync_copy(rows0_v, ptr_acc.at[pl.ds(blk * CHUNK, CHUNK)])
    plsc.subcore_barrier()

    rows_b = (rows0_v, rows1_v)
    rs_b = (rs0_v, rs1_v)
    tid_b = (tid0_v, tid1_v)
    tst_b = (tst0_v, tst1_v)
    en_b = (en0_v, en1_v)
    es_b = (es0_v, es1_v)
    sem_b = (sem0, sem1)
    ssem_b = (ssem0, ssem1)
    isem_b = (isem0, isem1)

    def _fire(j, b):
        pltpu.async_copy(vrepr_hbm.at[sid_ring.at[j % 4]], rows_b[b],
                         sem_b[b])
        pltpu.async_copy(tidx_hbm.at[wid, j], tid_b[b], sem_b[b])
        pltpu.async_copy(enorm_hbm.at[wid, j], en_b[b], sem_b[b])
        pltpu.async_copy(esgn_hbm.at[wid, j], es_b[b], sem_b[b])

    def _drain(b):
        pltpu.make_async_copy(vrepr_hbm.at[sid_ring.at[0]], rows_b[b],
                              sem_b[b]).wait()
        pltpu.make_async_copy(tidx_hbm.at[wid, 0], tid_b[b], sem_b[b]).wait()
        pltpu.make_async_copy(enorm_hbm.at[wid, 0], en_b[b], sem_b[b]).wait()
        pltpu.make_async_copy(esgn_hbm.at[wid, 0], es_b[b], sem_b[b]).wait()

    for c in range(4):
        pltpu.sync_copy(sidx_hbm.at[wid, c], sid_ring.at[c])
    _fire(0, 0)
    _fire(1, 1)

    def _drain_scatter(b):
        pltpu.make_async_copy(rs_b[b], ptr_acc.at[tst_b[b]],
                              ssem_b[b]).wait()

    def _pair(g, carry):
        for b in range(2):
            j = 2 * g + b

            @pl.when(j < NCHUNK)
            def _():
                _drain(b)

                @pl.when(j >= 2)
                def _():
                    _drain_scatter(b)

                rows_v = rows_b[b]
                rs_v = rs_b[b]
                en_v = en_b[b]
                es_v = es_b[b]

                for q in range(CHUNK // 16):
                    sl = pl.ds(q * 16, 16)
                    es_v[sl] = es_v[sl] * en_v[sl]
                    tst_b[b][sl] = tid_b[b][sl]

                for q in range(CHUNK // 16):
                    wv = es_v[pl.ds(q * 16, 16)]
                    for l in range(16):
                        i = q * 16 + l
                        w = wv[l]
                        for c8 in range(D // 16):
                            sl = pl.ds(c8 * 16, 16)
                            rs_v[i, sl] = rows_v[i, sl] * w

                pltpu.async_copy(rs_v, ptr_acc.at[tst_b[b]], ssem_b[b],
                                 add=True)

                @pl.when(j + 2 < NCHUNK)
                def _():
                    @pl.when(j >= 2)
                    def _():
                        pltpu.make_async_copy(sidx_hbm.at[wid, 0],
                                              sid_ring.at[0],
                                              isem_b[b]).wait()
                    _fire(j + 2, b)

                    @pl.when(j + 4 < NCHUNK)
                    def _():
                        pltpu.async_copy(sidx_hbm.at[wid, j + 4],
                                         sid_ring.at[(j + 4) % 4], isem_b[b])
        return carry
    lax.fori_loop(0, (NCHUNK + 1) // 2, _pair, 0)

    _drain_scatter(1)
    _drain_scatter(0)

    plsc.subcore_barrier()

    for b in range(BLK_ITERS):
        blk = sid + NS * b

        @pl.when(blk < NROWBLK)
        def _():
            pltpu.sync_copy(ptr_acc.at[pl.ds(blk * CHUNK, CHUNK)], rows0_v)
            pltpu.sync_copy(rows0_v,
                            out_hbm.at[cid, pl.ds(blk * CHUNK, CHUNK)])


def _segment_sum_sc(sidx_r, tidx_r, enorm_r, esgn_r, vrepr):
    mesh = plsc.VectorSubcoreMesh(core_axis_name="c", subcore_axis_name="s",
                                  num_cores=NC, num_subcores=NS)
    return pl.kernel(
        _sc_body,
        out_type=jax.ShapeDtypeStruct((NC, VNUM, D), jnp.float32),
        mesh=mesh,
        scratch_types=[
            pltpu.VMEM_SHARED((VNUM, D), jnp.float32),
            pltpu.VMEM((4, CHUNK), jnp.int32),
            pltpu.VMEM((CHUNK,), jnp.int32),
            pltpu.VMEM((CHUNK,), jnp.int32),
            pltpu.VMEM((CHUNK,), jnp.int32),
            pltpu.VMEM((CHUNK,), jnp.int32),
            pltpu.VMEM((CHUNK,), jnp.float32),
            pltpu.VMEM((CHUNK,), jnp.float32),
            pltpu.VMEM((CHUNK,), jnp.float32),
            pltpu.VMEM((CHUNK,), jnp.float32),
            pltpu.VMEM((CHUNK, D), jnp.float32),
            pltpu.VMEM((CHUNK, D), jnp.float32),
            pltpu.VMEM((CHUNK, D), jnp.float32),
            pltpu.VMEM((CHUNK, D), jnp.float32),
            pltpu.SemaphoreType.DMA,
            pltpu.SemaphoreType.DMA,
            pltpu.SemaphoreType.DMA,
            pltpu.SemaphoreType.DMA,
            pltpu.SemaphoreType.DMA,
            pltpu.SemaphoreType.DMA,
        ],
    )(sidx_r, tidx_r, enorm_r, esgn_r, vrepr)


def _tc_body(a_ref, b_ref, wl_ref, bl_ref, ws_ref, bs_ref, loc_ref, std_ref):
    ptr = a_ref[0] + b_ref[0]
    zl = jnp.dot(ptr, wl_ref[...], preferred_element_type=jnp.float32)
    zs = jnp.dot(ptr, ws_ref[...], preferred_element_type=jnp.float32)
    loc_ref[...] = zl + bl_ref[...]
    std_ref[...] = jax.nn.softplus(zs + bs_ref[...]) + EPS


def _heads_tc(partials, W_loc, b_loc, W_std, b_std):
    blk = 1000
    grid = (VNUM // blk,)
    return pl.pallas_call(
        _tc_body,
        grid=grid,
        in_specs=[
            pl.BlockSpec((1, blk, D), lambda i: (0, i, 0)),
            pl.BlockSpec((1, blk, D), lambda i: (1, i, 0)),
            pl.BlockSpec((D, D), lambda i: (0, 0)),
            pl.BlockSpec((D,), lambda i: (0,)),
            pl.BlockSpec((D, D), lambda i: (0, 0)),
            pl.BlockSpec((D,), lambda i: (0,)),
        ],
        out_specs=[
            pl.BlockSpec((blk, D), lambda i: (i, 0)),
            pl.BlockSpec((blk, D), lambda i: (i, 0)),
        ],
        out_shape=[
            jax.ShapeDtypeStruct((VNUM, D), jnp.float32),
            jax.ShapeDtypeStruct((VNUM, D), jnp.float32),
        ],
    )(partials, partials, W_loc, b_loc, W_std, b_std)


def kernel(eidx, enorm, esgn, vrepr, W_loc, b_loc, W_std, b_std):
    sidx_r = eidx[0].astype(jnp.int32).reshape(NW, NCHUNK, CHUNK)
    tidx_r = eidx[1].astype(jnp.int32).reshape(NW, NCHUNK, CHUNK)
    enorm_r = enorm.reshape(NW, NCHUNK, CHUNK)
    esgn_r = esgn.reshape(NW, NCHUNK, CHUNK)
    partials = _segment_sum_sc(sidx_r, tidx_r, enorm_r, esgn_r, vrepr)
    loc, std = _heads_tc(partials, W_loc, b_loc, W_std, b_std)
    return (loc, std)


# speedup vs baseline: 12.7804x; 1.1227x over previous
import functools

import jax
import jax.numpy as jnp
from jax import lax
from jax.experimental import pallas as pl
from jax.experimental.pallas import tpu as pltpu
from jax.experimental.pallas import tpu_sc as plsc

EPS = 1e-7

NC = 2
NS = 16
NW = NC * NS
VNUM = 10000
D = 128
E = 320000
EDGES_PER_W = E // NW
CHUNK = 80
NCHUNK = EDGES_PER_W // CHUNK
NROWBLK = VNUM // CHUNK
BLK_ITERS = -(-NROWBLK // NS)


def _sc_body(sidx_hbm, tidx_hbm, enorm_hbm, esgn_hbm, vrepr_hbm, out_hbm,
             ptr_acc, sid_ring, tid0_v, tid1_v, tst0_v, tst1_v,
             en0_v, en1_v, es0_v, es1_v,
             rows0_v, rows1_v, rs0_v, rs1_v,
             sem0, sem1, ssem0, ssem1, isem0, isem1):
    cid = lax.axis_index("c")
    sid = lax.axis_index("s")
    wid = cid * NS + sid

    def _zero_row(r, _):
        for c8 in range(D // 16):
            rs0_v[r, pl.ds(c8 * 16, 16)] = jnp.zeros((16,), jnp.float32)
        return _
    lax.fori_loop(0, CHUNK, _zero_row, 0)
    for b in range(BLK_ITERS):
        blk = sid + NS * b

        @pl.when(blk < NROWBLK)
        def _():
            pltpu.sync_copy(rs0_v, ptr_acc.at[pl.ds(blk * CHUNK, CHUNK)])
    plsc.subcore_barrier()

    rows_b = (rows0_v, rows1_v)
    rs_b = (rs0_v, rs1_v)
    tid_b = (tid0_v, tid1_v)
    tst_b = (tst0_v, tst1_v)
    en_b = (en0_v, en1_v)
    es_b = (es0_v, es1_v)
    sem_b = (sem0, sem1)
    ssem_b = (ssem0, ssem1)
    isem_b = (isem0, isem1)

    def _fire(j, b):
        pltpu.async_copy(vrepr_hbm.at[sid_ring.at[j % 4]], rows_b[b],
                         sem_b[b])
        pltpu.async_copy(tidx_hbm.at[wid, j], tid_b[b], sem_b[b])
        pltpu.async_copy(enorm_hbm.at[wid, j], en_b[b], sem_b[b])
        pltpu.async_copy(esgn_hbm.at[wid, j], es_b[b], sem_b[b])

    def _drain(b):
        pltpu.make_async_copy(vrepr_hbm.at[sid_ring.at[0]], rows_b[b],
                              sem_b[b]).wait()
        pltpu.make_async_copy(tidx_hbm.at[wid, 0], tid_b[b], sem_b[b]).wait()
        pltpu.make_async_copy(enorm_hbm.at[wid, 0], en_b[b], sem_b[b]).wait()
        pltpu.make_async_copy(esgn_hbm.at[wid, 0], es_b[b], sem_b[b]).wait()

    for c in range(4):
        pltpu.sync_copy(sidx_hbm.at[wid, c], sid_ring.at[c])
    _fire(0, 0)
    _fire(1, 1)

    def _drain_scatter(b):
        pltpu.make_async_copy(rs_b[b], ptr_acc.at[tst_b[b]],
                              ssem_b[b]).wait()

    def _pair(g, carry):
        for b in range(2):
            j = 2 * g + b

            @pl.when(j < NCHUNK)
            def _():
                _drain(b)

                @pl.when(j >= 2)
                def _():
                    _drain_scatter(b)

                rows_v = rows_b[b]
                rs_v = rs_b[b]
                en_v = en_b[b]
                es_v = es_b[b]

                for q in range(CHUNK // 16):
                    sl = pl.ds(q * 16, 16)
                    es_v[sl] = es_v[sl] * en_v[sl]
                    tst_b[b][sl] = tid_b[b][sl]

                for q in range(CHUNK // 16):
                    wv = es_v[pl.ds(q * 16, 16)]
                    for l in range(16):
                        i = q * 16 + l
                        w = wv[l]
                        for cb in range(4):
                            x = rows_v[i, pl.ds(cb * 16, 16)]
                            a = lax.bitcast_convert_type(
                                x << 16, jnp.float32)
                            b2 = lax.bitcast_convert_type(
                                x & jnp.int32(-65536), jnp.float32)
                            rs_v[i, pl.ds(cb * 32, 16)] = a * w
                            rs_v[i, pl.ds(cb * 32 + 16, 16)] = b2 * w

                pltpu.async_copy(rs_v, ptr_acc.at[tst_b[b]], ssem_b[b],
                                 add=True)

                @pl.when(j + 2 < NCHUNK)
                def _():
                    @pl.when(j >= 2)
                    def _():
                        pltpu.make_async_copy(sidx_hbm.at[wid, 0],
                                              sid_ring.at[0],
                                              isem_b[b]).wait()
                    _fire(j + 2, b)

                    @pl.when(j + 4 < NCHUNK)
                    def _():
                        pltpu.async_copy(sidx_hbm.at[wid, j + 4],
                                         sid_ring.at[(j + 4) % 4], isem_b[b])
        return carry
    lax.fori_loop(0, (NCHUNK + 1) // 2, _pair, 0)

    _drain_scatter(1)
    _drain_scatter(0)

    plsc.subcore_barrier()

    for b in range(BLK_ITERS):
        blk = sid + NS * b

        @pl.when(blk < NROWBLK)
        def _():
            pltpu.sync_copy(ptr_acc.at[pl.ds(blk * CHUNK, CHUNK)], rs0_v)
            pltpu.sync_copy(rs0_v,
                            out_hbm.at[cid, pl.ds(blk * CHUNK, CHUNK)])


def _segment_sum_sc(sidx_r, tidx_r, enorm_r, esgn_r, vrepr):
    mesh = plsc.VectorSubcoreMesh(core_axis_name="c", subcore_axis_name="s",
                                  num_cores=NC, num_subcores=NS)
    return pl.kernel(
        _sc_body,
        out_type=jax.ShapeDtypeStruct((NC, VNUM, D), jnp.float32),
        mesh=mesh,
        compiler_params=pltpu.CompilerParams(use_tc_tiling_on_sc=False),
        scratch_types=[
            pltpu.VMEM_SHARED((VNUM, D), jnp.float32),
            pltpu.VMEM((4, CHUNK), jnp.int32),
            pltpu.VMEM((CHUNK,), jnp.int32),
            pltpu.VMEM((CHUNK,), jnp.int32),
            pltpu.VMEM((CHUNK,), jnp.int32),
            pltpu.VMEM((CHUNK,), jnp.int32),
            pltpu.VMEM((CHUNK,), jnp.float32),
            pltpu.VMEM((CHUNK,), jnp.float32),
            pltpu.VMEM((CHUNK,), jnp.float32),
            pltpu.VMEM((CHUNK,), jnp.float32),
            pltpu.VMEM((CHUNK, D // 2), jnp.int32),
            pltpu.VMEM((CHUNK, D // 2), jnp.int32),
            pltpu.VMEM((CHUNK, D), jnp.float32),
            pltpu.VMEM((CHUNK, D), jnp.float32),
            pltpu.SemaphoreType.DMA,
            pltpu.SemaphoreType.DMA,
            pltpu.SemaphoreType.DMA,
            pltpu.SemaphoreType.DMA,
            pltpu.SemaphoreType.DMA,
            pltpu.SemaphoreType.DMA,
        ],
    )(sidx_r, tidx_r, enorm_r, esgn_r, vrepr)


def _tc_body(a_ref, b_ref, wl_ref, bl_ref, ws_ref, bs_ref, loc_ref, std_ref):
    ptr = a_ref[0] + b_ref[0]
    zl = jnp.dot(ptr, wl_ref[...], preferred_element_type=jnp.float32)
    zs = jnp.dot(ptr, ws_ref[...], preferred_element_type=jnp.float32)
    loc_ref[...] = zl + bl_ref[...]
    std_ref[...] = jax.nn.softplus(zs + bs_ref[...]) + EPS


def _heads_tc(partials, W_loc, b_loc, W_std, b_std):
    blk = 1000
    grid = (VNUM // blk,)
    return pl.pallas_call(
        _tc_body,
        grid=grid,
        in_specs=[
            pl.BlockSpec((1, blk, D), lambda i: (0, i, 0)),
            pl.BlockSpec((1, blk, D), lambda i: (1, i, 0)),
            pl.BlockSpec((D, D), lambda i: (0, 0)),
            pl.BlockSpec((D,), lambda i: (0,)),
            pl.BlockSpec((D, D), lambda i: (0, 0)),
            pl.BlockSpec((D,), lambda i: (0,)),
        ],
        out_specs=[
            pl.BlockSpec((blk, D), lambda i: (i, 0)),
            pl.BlockSpec((blk, D), lambda i: (i, 0)),
        ],
        out_shape=[
            jax.ShapeDtypeStruct((VNUM, D), jnp.float32),
            jax.ShapeDtypeStruct((VNUM, D), jnp.float32),
        ],
    )(partials, partials, W_loc, b_loc, W_std, b_std)


def kernel(eidx, enorm, esgn, vrepr, W_loc, b_loc, W_std, b_std):
    sidx_r = eidx[0].astype(jnp.int32).reshape(NW, NCHUNK, CHUNK)
    tidx_r = eidx[1].astype(jnp.int32).reshape(NW, NCHUNK, CHUNK)
    enorm_r = enorm.reshape(NW, NCHUNK, CHUNK)
    esgn_r = esgn.reshape(NW, NCHUNK, CHUNK)
    vshuf = vrepr.reshape(VNUM, 4, 2, 16).swapaxes(2, 3).reshape(VNUM, 64, 2)
    vtab = jax.lax.bitcast_convert_type(vshuf.astype(jnp.bfloat16),
                                        jnp.int32)
    partials = _segment_sum_sc(sidx_r, tidx_r, enorm_r, esgn_r, vtab)
    loc, std = _heads_tc(partials, W_loc, b_loc, W_std, b_std)
    return (loc, std)
